# baseline (speedup 1.0000x reference)
import functools

import jax
import jax.numpy as jnp
from jax import lax
from jax.experimental import pallas as pl
from jax.experimental.pallas import tpu as pltpu
from jax.experimental.pallas import tpu_sc as plsc

NC, NS = 2, 16
NW = NC * NS
B = 4096 * 200
D = 64
BPW = B // NW
CH = 800
NCHUNK = BPW // CH

_MESH = plsc.VectorSubcoreMesh(
    core_axis_name="c", subcore_axis_name="s", num_cores=NC, num_subcores=NS
)


@functools.partial(
    pl.kernel,
    out_type=jax.ShapeDtypeStruct((B, D), jnp.float32),
    mesh=_MESH,
    compiler_params=pltpu.CompilerParams(use_tc_tiling_on_sc=False),
    scratch_types=[
        pltpu.VMEM((BPW,), jnp.int32),
        pltpu.VMEM((CH, D), jnp.float32),
        pltpu.SemaphoreType.DMA,
    ],
)
def _gather(x_hbm, table_hbm, out_hbm, idx_v, rows, sem):
    wid = lax.axis_index("s") * NC + lax.axis_index("c")
    base = wid * BPW
    pltpu.sync_copy(x_hbm.at[pl.ds(base, BPW)], idx_v)

    @pl.loop(0, NCHUNK)
    def _chunk(g):
        off = g * CH
        pltpu.async_copy(
            table_hbm.at[idx_v.at[pl.ds(off, CH)]], rows, sem
        ).wait()
        pltpu.sync_copy(rows, out_hbm.at[pl.ds(base + off, CH)])


def kernel(x, table):
    out = _gather(x.reshape(-1), table)
    return out.reshape(x.shape + (D,))


# speedup vs baseline: 1.0133x; 1.0133x over previous
import functools

import jax
import jax.numpy as jnp
from jax import lax
from jax.experimental import pallas as pl
from jax.experimental.pallas import tpu as pltpu
from jax.experimental.pallas import tpu_sc as plsc

NC, NS = 2, 16
NW = NC * NS
B = 4096 * 200
D = 64
BPW = B // NW
CH = 800
NCHUNK = BPW // CH

_MESH = plsc.VectorSubcoreMesh(
    core_axis_name="c", subcore_axis_name="s", num_cores=NC, num_subcores=NS
)


@functools.partial(
    pl.kernel,
    out_type=jax.ShapeDtypeStruct((B, D), jnp.float32),
    mesh=_MESH,
    compiler_params=pltpu.CompilerParams(use_tc_tiling_on_sc=False),
    scratch_types=[
        pltpu.VMEM((BPW,), jnp.int32),
        pltpu.VMEM((CH, D), jnp.float32),
        pltpu.VMEM((CH, D), jnp.float32),
        pltpu.SemaphoreType.DMA,
        pltpu.SemaphoreType.DMA,
    ],
)
def _gather(x_hbm, table_hbm, out_hbm, idx_v, rows0, rows1, sem0, sem1):
    wid = lax.axis_index("s") * NC + lax.axis_index("c")
    base = wid * BPW
    pltpu.sync_copy(x_hbm.at[pl.ds(base, BPW)], idx_v)
    rows = (rows0, rows1)
    sems = (sem0, sem1)

    def start(g, b):
        pltpu.async_copy(table_hbm.at[idx_v.at[pl.ds(g * CH, CH)]], rows[b], sems[b])

    def finish(g, b):
        pltpu.make_async_copy(
            table_hbm.at[idx_v.at[pl.ds(g * CH, CH)]], rows[b], sems[b]
        ).wait()
        pltpu.sync_copy(rows[b], out_hbm.at[pl.ds(base + g * CH, CH)])

    start(0, 0)

    @pl.loop(0, NCHUNK, step=2)
    def _chunk(g):
        start(g + 1, 1)
        finish(g, 0)

        @pl.when(g + 2 < NCHUNK)
        def _():
            start(g + 2, 0)

        finish(g + 1, 1)


def kernel(x, table):
    out = _gather(x.reshape(-1), table)
    return out.reshape(x.shape + (D,))
